# baseline (speedup 1.0000x reference)
import functools

import jax
import jax.numpy as jnp
from jax import lax
from jax.experimental import pallas as pl
from jax.experimental.pallas import tpu as pltpu
from jax.experimental.pallas import tpu_sc as plsc

_D = 512
_SEGS = 512
_SEG_LEN = 32
_TOTAL = _SEGS * _SEG_LEN
_LANES = 16

_SC_ROWS = 16
_TC_ROWS = _D - _SC_ROWS

_NW = 32
_ELEMS = _SC_ROWS * _TOTAL
_OUT = _SC_ROWS * _SEGS
_ELEMS_W = _ELEMS // _NW
_OUT_W = _OUT // _NW
_CHUNK = 8192
_NCHUNK = _ELEMS_W // _CHUNK
_NBUF = 1
_GROUPS = _CHUNK // (_LANES * _SEG_LEN)


def _make_sc_kernel():
    info = plsc.get_sparse_core_info()
    nc = info.num_cores
    mesh = plsc.VectorSubcoreMesh(core_axis_name="c", subcore_axis_name="s")

    @functools.partial(
        pl.kernel,
        out_type=jax.ShapeDtypeStruct((_OUT,), jnp.float32),
        mesh=mesh,
        scratch_types=(
            [pltpu.VMEM((_CHUNK,), jnp.float32) for _ in range(_NBUF)]
            + [pltpu.VMEM((_OUT_W,), jnp.float32)]
            + [pltpu.SemaphoreType.DMA for _ in range(_NBUF)]
        ),
        compiler_params=pltpu.CompilerParams(
            needs_layout_passes=False, skip_device_barrier=True),
    )
    def prod_at(x_hbm, out_hbm, *refs):
        bufs = refs[:_NBUF]
        out_v = refs[_NBUF]
        sems = refs[_NBUF + 1:]
        wid = lax.axis_index("s") * nc + lax.axis_index("c")
        in_base = wid * _ELEMS_W
        lane_iota = lax.broadcasted_iota(jnp.int32, (_LANES,), 0)
        rot_idx = [
            lane_iota * _SEG_LEN + ((lane_iota + i) & (_SEG_LEN - 1))
            for i in range(_SEG_LEN)
        ]

        handles = [None] * _NBUF

        def start(c):
            b = c % _NBUF
            handles[b] = pltpu.async_copy(
                x_hbm.at[pl.ds(in_base + c * _CHUNK, _CHUNK)],
                bufs[b], sems[b])

        for c in range(_NBUF - 1):
            start(c)
        for c in range(_NCHUNK):
            b = c % _NBUF
            if c + _NBUF - 1 < _NCHUNK:
                start(c + _NBUF - 1)
            handles[b].wait()
            buf = bufs[b]
            out_off = c * (_GROUPS * _LANES)

            def grp_body(g, carry, buf=buf, out_off=out_off):
                base = g * (_LANES * _SEG_LEN)
                vals = [
                    plsc.load_gather(buf, [rot_idx[i] + base])
                    for i in range(_SEG_LEN)
                ]
                while len(vals) > 1:
                    vals = [a * b for a, b in zip(vals[::2], vals[1::2])]
                out_v[pl.ds(out_off + g * _LANES, _LANES)] = vals[0]
                return carry

            lax.fori_loop(0, _GROUPS, grp_body, 0)

        pltpu.sync_copy(out_v, out_hbm.at[pl.ds(wid * _OUT_W, _OUT_W)])

    return prod_at


_sc_kernel = _make_sc_kernel()

_CC = 4096
_NSEG_CC = _CC // _SEG_LEN


def _tc_body(x_ref, o_ref):
    t = lax.broadcasted_iota(jnp.int32, (_CC, _NSEG_CC), 0)
    s = lax.broadcasted_iota(jnp.int32, (_CC, _NSEG_CC), 1)
    onehot = (t // _SEG_LEN == s).astype(jnp.float32)
    lx = jnp.log(x_ref[...])
    acc = lax.dot_general(
        lx, onehot, (((1,), (0,)), ((), ())),
        preferred_element_type=jnp.float32)
    o_ref[...] = jnp.exp(acc)


_tc_part = pl.pallas_call(
    _tc_body,
    grid=(_TOTAL // _CC,),
    in_specs=[pl.BlockSpec((_TC_ROWS, _CC), lambda j: (0, j))],
    out_specs=pl.BlockSpec((_TC_ROWS, _NSEG_CC), lambda j: (0, j)),
    out_shape=jax.ShapeDtypeStruct((_TC_ROWS, _SEGS), jnp.float32),
)


def kernel(x):
    sc_out = _sc_kernel(x[_TC_ROWS:].reshape(_ELEMS))
    tc_out = _tc_part(x)
    return jnp.concatenate([tc_out, sc_out.reshape(_SC_ROWS, _SEGS)], axis=0)


# speedup vs baseline: 1.0174x; 1.0174x over previous
import functools

import jax
import jax.numpy as jnp
from jax import lax
from jax.experimental import pallas as pl
from jax.experimental.pallas import tpu as pltpu
from jax.experimental.pallas import tpu_sc as plsc

_D = 512
_SEGS = 512
_SEG_LEN = 32
_TOTAL = _SEGS * _SEG_LEN
_LANES = 16

_SC_ROWS = 8
_TC_ROWS = _D - _SC_ROWS

_NW = 32
_ELEMS = _SC_ROWS * _TOTAL
_OUT = _SC_ROWS * _SEGS
_ELEMS_W = _ELEMS // _NW
_OUT_W = _OUT // _NW
_CHUNK = 4096
_NCHUNK = _ELEMS_W // _CHUNK
_NBUF = 1
_GROUPS = _CHUNK // (_LANES * _SEG_LEN)


def _make_sc_kernel():
    info = plsc.get_sparse_core_info()
    nc = info.num_cores
    mesh = plsc.VectorSubcoreMesh(core_axis_name="c", subcore_axis_name="s")

    @functools.partial(
        pl.kernel,
        out_type=jax.ShapeDtypeStruct((_OUT,), jnp.float32),
        mesh=mesh,
        scratch_types=(
            [pltpu.VMEM((_CHUNK,), jnp.float32) for _ in range(_NBUF)]
            + [pltpu.VMEM((_OUT_W,), jnp.float32)]
            + [pltpu.SemaphoreType.DMA for _ in range(_NBUF)]
        ),
        compiler_params=pltpu.CompilerParams(needs_layout_passes=False),
    )
    def prod_at(x_hbm, out_hbm, *refs):
        bufs = refs[:_NBUF]
        out_v = refs[_NBUF]
        sems = refs[_NBUF + 1:]
        wid = lax.axis_index("s") * nc + lax.axis_index("c")
        in_base = wid * _ELEMS_W
        lane_iota = lax.broadcasted_iota(jnp.int32, (_LANES,), 0)
        rot_idx = [
            lane_iota * _SEG_LEN + ((lane_iota + i) & (_SEG_LEN - 1))
            for i in range(_SEG_LEN)
        ]

        handles = [None] * _NBUF

        def start(c):
            b = c % _NBUF
            handles[b] = pltpu.async_copy(
                x_hbm.at[pl.ds(in_base + c * _CHUNK, _CHUNK)],
                bufs[b], sems[b])

        for c in range(_NBUF - 1):
            start(c)
        for c in range(_NCHUNK):
            b = c % _NBUF
            if c + _NBUF - 1 < _NCHUNK:
                start(c + _NBUF - 1)
            handles[b].wait()
            buf = bufs[b]
            out_off = c * (_GROUPS * _LANES)

            def grp_body(g, carry, buf=buf, out_off=out_off):
                base = g * (_LANES * _SEG_LEN)
                vals = [
                    plsc.load_gather(buf, [rot_idx[i] + base])
                    for i in range(_SEG_LEN)
                ]
                while len(vals) > 1:
                    vals = [a * b for a, b in zip(vals[::2], vals[1::2])]
                out_v[pl.ds(out_off + g * _LANES, _LANES)] = vals[0]
                return carry

            lax.fori_loop(0, _GROUPS, grp_body, 0)

        pltpu.sync_copy(out_v, out_hbm.at[pl.ds(wid * _OUT_W, _OUT_W)])

    return prod_at


_sc_kernel = _make_sc_kernel()

_CC = 4096
_NSEG_CC = _CC // _SEG_LEN


def _tc_body(x_ref, o_ref):
    t = lax.broadcasted_iota(jnp.int32, (_CC, _NSEG_CC), 0)
    s = lax.broadcasted_iota(jnp.int32, (_CC, _NSEG_CC), 1)
    onehot = (t // _SEG_LEN == s).astype(jnp.float32)
    lx = jnp.log(x_ref[...])
    acc = lax.dot_general(
        lx, onehot, (((1,), (0,)), ((), ())),
        preferred_element_type=jnp.float32)
    o_ref[...] = jnp.exp(acc)


_tc_part = pl.pallas_call(
    _tc_body,
    grid=(_TOTAL // _CC,),
    in_specs=[pl.BlockSpec((_TC_ROWS, _CC), lambda j: (0, j))],
    out_specs=pl.BlockSpec((_TC_ROWS, _NSEG_CC), lambda j: (0, j)),
    out_shape=jax.ShapeDtypeStruct((_TC_ROWS, _SEGS), jnp.float32),
)


def kernel(x):
    sc_out = _sc_kernel(x[_TC_ROWS:].reshape(_ELEMS))
    tc_out = _tc_part(x)
    return jnp.concatenate([tc_out, sc_out.reshape(_SC_ROWS, _SEGS)], axis=0)
